# baseline (speedup 1.0000x reference)
import functools

import jax
import jax.numpy as jnp
from jax import lax
from jax.experimental import pallas as pl
from jax.experimental.pallas import tpu as pltpu
from jax.experimental.pallas import tpu_sc as plsc

B = 16384
F = 32
NC, NS, L = 2, 16, 16
NW = NC * NS
BPW = B // NW
W = 128
K = 4


def _mf_body(user_hbm, item_hbm, uft_hbm, ift_hbm, out_hbm,
             uidx_v, iidx_v, uwA, vwA, uwB, vwB, outv, semA, semB):
    wid = lax.axis_index("s") * NC + lax.axis_index("c")
    base = wid * BPW

    pltpu.sync_copy(user_hbm.at[pl.ds(base, BPW)], uidx_v)
    pltpu.sync_copy(item_hbm.at[pl.ds(base, BPW)], iidx_v)

    iota = lax.iota(jnp.int32, L)
    bufs = [(uwA, vwA, semA), (uwB, vwB, semB)]

    def step(j, carry):
        uvec = uidx_v[pl.ds(j * L, L)]
        ivec = iidx_v[pl.ds(j * L, L)]
        uh = (uvec // W) * W
        ih = (ivec // W) * W
        uq = uvec - uh
        iq = ivec - ih

        def fire(sg):
            ub, vb, sem = bufs[sg & 1]
            cps = []
            for k in range(K):
                lane = sg * K + k
                cps.append(pltpu.async_copy(
                    uft_hbm.at[:, pl.ds(pl.multiple_of(uh[lane], W), W)],
                    ub.at[k], sem))
                cps.append(pltpu.async_copy(
                    ift_hbm.at[:, pl.ds(pl.multiple_of(ih[lane], W), W)],
                    vb.at[k], sem))
            return cps

        def compute(sg, res):
            ub, vb, _ = bufs[sg & 1]
            for k in range(K):
                lane = sg * K + k
                slot = jnp.full((L,), k, jnp.int32)
                uql = jnp.full((L,), uq[lane], jnp.int32)
                iql = jnp.full((L,), iq[lane], jnp.int32)
                ulo = plsc.load_gather(ub, [slot, iota, uql])
                uhi = plsc.load_gather(ub, [slot, iota + L, uql])
                vlo = plsc.load_gather(vb, [slot, iota, iql])
                vhi = plsc.load_gather(vb, [slot, iota + L, iql])
                dot = jnp.sum(ulo * vlo + uhi * vhi, axis=0)
                res = jnp.where(iota == lane, dot, res)
            return res

        res = jnp.zeros((L,), jnp.float32)
        pend = [fire(0), fire(1)]
        for sg in range(L // K):
            for cp in pend.pop(0):
                cp.wait()
            res = compute(sg, res)
            if sg + 2 < L // K:
                pend.append(fire(sg + 2))
        outv[pl.ds(j * L, L)] = res
        return carry

    lax.fori_loop(0, BPW // L, step, 0)

    pltpu.sync_copy(outv, out_hbm.at[pl.ds(base, BPW)])


_mf = functools.partial(
    pl.kernel,
    out_type=jax.ShapeDtypeStruct((B,), jnp.float32),
    mesh=plsc.VectorSubcoreMesh(core_axis_name="c", subcore_axis_name="s",
                                num_cores=NC, num_subcores=NS),
    compiler_params=pltpu.CompilerParams(needs_layout_passes=False),
    scratch_types=[
        pltpu.VMEM((BPW,), jnp.int32),
        pltpu.VMEM((BPW,), jnp.int32),
        pltpu.VMEM((K, F, W), jnp.float32),
        pltpu.VMEM((K, F, W), jnp.float32),
        pltpu.VMEM((K, F, W), jnp.float32),
        pltpu.VMEM((K, F, W), jnp.float32),
        pltpu.VMEM((BPW,), jnp.float32),
        pltpu.SemaphoreType.DMA,
        pltpu.SemaphoreType.DMA,
    ],
)(_mf_body)


def kernel(user, item, user_factors, item_factors):
    return _mf(user.astype(jnp.int32), item.astype(jnp.int32),
               user_factors.T, item_factors.T)


# speedup vs baseline: 1.0270x; 1.0270x over previous
import functools

import jax
import jax.numpy as jnp
from jax import lax
from jax.experimental import pallas as pl
from jax.experimental.pallas import tpu as pltpu
from jax.experimental.pallas import tpu_sc as plsc

B = 16384
F = 32
NC, NS, L = 2, 16, 16
NW = NC * NS
BPW = B // NW
W = 128
K = 8


def _mf_body(user_hbm, item_hbm, uft_hbm, ift_hbm, out_hbm,
             uidx_v, iidx_v, uwins, vwins, outv, sem):
    wid = lax.axis_index("s") * NC + lax.axis_index("c")
    base = wid * BPW

    pltpu.sync_copy(user_hbm.at[pl.ds(base, BPW)], uidx_v)
    pltpu.sync_copy(item_hbm.at[pl.ds(base, BPW)], iidx_v)

    iota = lax.iota(jnp.int32, L)

    def step(j, carry):
        uvec = uidx_v[pl.ds(j * L, L)]
        ivec = iidx_v[pl.ds(j * L, L)]
        uh = (uvec // W) * W
        ih = (ivec // W) * W
        uq = uvec - uh
        iq = ivec - ih
        res = jnp.zeros((L,), jnp.float32)
        for half in range(L // K):
            copies = []
            for k in range(K):
                lane = half * K + k
                copies.append(pltpu.async_copy(
                    uft_hbm.at[:, pl.ds(pl.multiple_of(uh[lane], W), W)],
                    uwins.at[k], sem))
                copies.append(pltpu.async_copy(
                    ift_hbm.at[:, pl.ds(pl.multiple_of(ih[lane], W), W)],
                    vwins.at[k], sem))
            for cp in copies:
                cp.wait()
            for k in range(K):
                lane = half * K + k
                slot = jnp.full((L,), k, jnp.int32)
                uql = jnp.full((L,), uq[lane], jnp.int32)
                iql = jnp.full((L,), iq[lane], jnp.int32)
                ulo = plsc.load_gather(uwins, [slot, iota, uql])
                uhi = plsc.load_gather(uwins, [slot, iota + L, uql])
                vlo = plsc.load_gather(vwins, [slot, iota, iql])
                vhi = plsc.load_gather(vwins, [slot, iota + L, iql])
                dot = jnp.sum(ulo * vlo + uhi * vhi, axis=0)
                res = jnp.where(iota == lane, dot, res)
        outv[pl.ds(j * L, L)] = res
        return carry

    lax.fori_loop(0, BPW // L, step, 0)

    pltpu.sync_copy(outv, out_hbm.at[pl.ds(base, BPW)])


_mf = functools.partial(
    pl.kernel,
    out_type=jax.ShapeDtypeStruct((B,), jnp.float32),
    mesh=plsc.VectorSubcoreMesh(core_axis_name="c", subcore_axis_name="s",
                                num_cores=NC, num_subcores=NS),
    compiler_params=pltpu.CompilerParams(needs_layout_passes=False),
    scratch_types=[
        pltpu.VMEM((BPW,), jnp.int32),
        pltpu.VMEM((BPW,), jnp.int32),
        pltpu.VMEM((K, F, W), jnp.float32),
        pltpu.VMEM((K, F, W), jnp.float32),
        pltpu.VMEM((BPW,), jnp.float32),
        pltpu.SemaphoreType.DMA,
    ],
)(_mf_body)


def kernel(user, item, user_factors, item_factors):
    return _mf(user.astype(jnp.int32), item.astype(jnp.int32),
               user_factors.T, item_factors.T)
